# baseline (speedup 1.0000x reference)
import functools

import jax
import jax.numpy as jnp
from jax.experimental import pallas as pl
from jax.experimental.pallas import tpu as pltpu


def _pass_a_body(x_ref, adj_ref, W1_ref, b1_ref, W2_ref, q_ref, s2q_ref,
                 sc_ref, s1_ref, h_ref, *, tm, nt):
    i = pl.program_id(0)

    @pl.when(i == 0)
    def _():
        s1_ref[...] = jnp.dot(x_ref[...], W1_ref[...],
                              preferred_element_type=jnp.float32)

    a = adj_ref[...]
    acc = jnp.dot(a, s1_ref[...], preferred_element_type=jnp.float32)
    h_ref[pl.ds(i * tm, tm), :] = jnp.maximum(acc + b1_ref[...], 0.0)
    q_ref[...] = (a - 0.5).astype(jnp.float8_e4m3fn)

    @pl.when(i == nt - 1)
    def _():
        s2t = jax.lax.dot_general(
            W2_ref[...], h_ref[...], (((0,), (1,)), ((), ())),
            preferred_element_type=jnp.float32)
        sc1 = jnp.maximum(jnp.max(jnp.abs(s2t), axis=1, keepdims=True),
                          1e-30) * (1.0 / 240.0)
        q1 = (s2t / sc1).astype(jnp.float8_e4m3fn)
        r = s2t - q1.astype(jnp.float32) * sc1
        sc2 = jnp.maximum(jnp.max(jnp.abs(r), axis=1, keepdims=True),
                          1e-30) * (1.0 / 240.0)
        q2 = (r / sc2).astype(jnp.float8_e4m3fn)
        s2q_ref[:, : q1.shape[0]] = q1.T
        s2q_ref[:, q1.shape[0]:] = q2.T
        cs1 = jnp.sum(q1.astype(jnp.float32), axis=1, keepdims=True)
        cs2 = jnp.sum(q2.astype(jnp.float32), axis=1, keepdims=True)
        sc_ref[0:1, :] = sc1.T
        sc_ref[1:2, :] = sc2.T
        sc_ref[2:3, :] = 0.5 * (cs1 * sc1 + cs2 * sc2).T


def _pass_b_body(q_ref, s2q_ref, sc_ref, b2_ref, out_ref, *, nclass):
    acc = jnp.dot(q_ref[...], s2q_ref[...],
                  preferred_element_type=jnp.float32)
    o = (acc[:, :nclass] * sc_ref[0:1, :]
         + acc[:, nclass:] * sc_ref[1:2, :]
         + sc_ref[2:3, :] + b2_ref[...])
    m = jnp.max(o, axis=1, keepdims=True)
    lse = jnp.log(jnp.sum(jnp.exp(o - m), axis=1, keepdims=True)) + m
    out_ref[...] = o - lse


def kernel(x, adj, W1, b1, W2, b2):
    n, nfeat = x.shape
    nhid = W1.shape[1]
    nclass = W2.shape[1]
    tm_a = 400
    nt_a = n // tm_a
    tm_b = 1000
    nt_b = n // tm_b

    b1r = b1.reshape(1, nhid)
    b2r = b2.reshape(1, nclass)

    q, s2q, sc = pl.pallas_call(
        functools.partial(_pass_a_body, tm=tm_a, nt=nt_a),
        grid=(nt_a,),
        in_specs=[
            pl.BlockSpec((n, nfeat), lambda i: (0, 0)),
            pl.BlockSpec((tm_a, n), lambda i: (i, 0)),
            pl.BlockSpec((nfeat, nhid), lambda i: (0, 0)),
            pl.BlockSpec((1, nhid), lambda i: (0, 0)),
            pl.BlockSpec((nhid, nclass), lambda i: (0, 0)),
        ],
        out_specs=[
            pl.BlockSpec((tm_a, n), lambda i: (i, 0)),
            pl.BlockSpec((n, 2 * nclass), lambda i: (0, 0)),
            pl.BlockSpec((3, nclass), lambda i: (0, 0)),
        ],
        out_shape=[
            jax.ShapeDtypeStruct((n, n), jnp.float8_e4m3fn),
            jax.ShapeDtypeStruct((n, 2 * nclass), jnp.float8_e4m3fn),
            jax.ShapeDtypeStruct((3, nclass), jnp.float32),
        ],
        scratch_shapes=[
            pltpu.VMEM((n, nhid), jnp.float32),
            pltpu.VMEM((n, nhid), jnp.float32),
        ],
        compiler_params=pltpu.CompilerParams(
            dimension_semantics=("arbitrary",),
            vmem_limit_bytes=63 * 1024 * 1024,
        ),
    )(x, adj, W1, b1r, W2)

    return pl.pallas_call(
        functools.partial(_pass_b_body, nclass=nclass),
        grid=(nt_b,),
        in_specs=[
            pl.BlockSpec((tm_b, n), lambda i: (i, 0)),
            pl.BlockSpec((n, 2 * nclass), lambda i: (0, 0)),
            pl.BlockSpec((3, nclass), lambda i: (0, 0)),
            pl.BlockSpec((1, nclass), lambda i: (0, 0)),
        ],
        out_specs=pl.BlockSpec((tm_b, nclass), lambda i: (i, 0)),
        out_shape=jax.ShapeDtypeStruct((n, nclass), jnp.float32),
        compiler_params=pltpu.CompilerParams(
            dimension_semantics=("arbitrary",),
        ),
    )(q, s2q, sc, b2r)


# speedup vs baseline: 1.0015x; 1.0015x over previous
import functools

import jax
import jax.numpy as jnp
from jax.experimental import pallas as pl
from jax.experimental.pallas import tpu as pltpu


def _pass_a_body(x_ref, adj_ref, W1_ref, b1_ref, W2_ref, q_ref, s2q_ref,
                 sc_ref, s1_ref, h_ref, *, tm, nt):
    i = pl.program_id(0)

    @pl.when(i == 0)
    def _():
        s1_ref[...] = jnp.dot(x_ref[...], W1_ref[...],
                              preferred_element_type=jnp.float32)

    a = adj_ref[...]
    acc = jnp.dot(a, s1_ref[...], preferred_element_type=jnp.float32)
    h_ref[pl.ds(i * tm, tm), :] = jnp.maximum(acc + b1_ref[...], 0.0)
    q_ref[...] = a.astype(jnp.float8_e4m3fn)

    @pl.when(i == nt - 1)
    def _():
        s2t = jax.lax.dot_general(
            W2_ref[...], h_ref[...], (((0,), (1,)), ((), ())),
            preferred_element_type=jnp.float32)
        sc1 = jnp.maximum(jnp.max(jnp.abs(s2t), axis=1, keepdims=True),
                          1e-30) * (1.0 / 240.0)
        q1 = (s2t / sc1).astype(jnp.float8_e4m3fn)
        r = s2t - q1.astype(jnp.float32) * sc1
        sc2 = jnp.maximum(jnp.max(jnp.abs(r), axis=1, keepdims=True),
                          1e-30) * (1.0 / 240.0)
        q2 = (r / sc2).astype(jnp.float8_e4m3fn)
        s2q_ref[:, : q1.shape[0]] = q1.T
        s2q_ref[:, q1.shape[0]:] = q2.T
        sc_ref[0:1, :] = sc1.T
        sc_ref[1:2, :] = sc2.T


def _pass_b_body(q_ref, s2q_ref, sc_ref, b2_ref, out_ref, *, nclass):
    acc = jnp.dot(q_ref[...], s2q_ref[...],
                  preferred_element_type=jnp.float32)
    o = (acc[:, :nclass] * sc_ref[0:1, :]
         + acc[:, nclass:] * sc_ref[1:2, :]
         + b2_ref[...])
    m = jnp.max(o, axis=1, keepdims=True)
    lse = jnp.log(jnp.sum(jnp.exp(o - m), axis=1, keepdims=True)) + m
    out_ref[...] = o - lse


def kernel(x, adj, W1, b1, W2, b2):
    n, nfeat = x.shape
    nhid = W1.shape[1]
    nclass = W2.shape[1]
    tm_a = 400
    nt_a = n // tm_a
    tm_b = 1000
    nt_b = n // tm_b

    b1r = b1.reshape(1, nhid)
    b2r = b2.reshape(1, nclass)

    q, s2q, sc = pl.pallas_call(
        functools.partial(_pass_a_body, tm=tm_a, nt=nt_a),
        grid=(nt_a,),
        in_specs=[
            pl.BlockSpec((n, nfeat), lambda i: (0, 0)),
            pl.BlockSpec((tm_a, n), lambda i: (i, 0)),
            pl.BlockSpec((nfeat, nhid), lambda i: (0, 0)),
            pl.BlockSpec((1, nhid), lambda i: (0, 0)),
            pl.BlockSpec((nhid, nclass), lambda i: (0, 0)),
        ],
        out_specs=[
            pl.BlockSpec((tm_a, n), lambda i: (i, 0)),
            pl.BlockSpec((n, 2 * nclass), lambda i: (0, 0)),
            pl.BlockSpec((2, nclass), lambda i: (0, 0)),
        ],
        out_shape=[
            jax.ShapeDtypeStruct((n, n), jnp.float8_e4m3fn),
            jax.ShapeDtypeStruct((n, 2 * nclass), jnp.float8_e4m3fn),
            jax.ShapeDtypeStruct((2, nclass), jnp.float32),
        ],
        scratch_shapes=[
            pltpu.VMEM((n, nhid), jnp.float32),
            pltpu.VMEM((n, nhid), jnp.float32),
        ],
        compiler_params=pltpu.CompilerParams(
            dimension_semantics=("arbitrary",),
            vmem_limit_bytes=63 * 1024 * 1024,
        ),
    )(x, adj, W1, b1r, W2)

    return pl.pallas_call(
        functools.partial(_pass_b_body, nclass=nclass),
        grid=(nt_b,),
        in_specs=[
            pl.BlockSpec((tm_b, n), lambda i: (i, 0)),
            pl.BlockSpec((n, 2 * nclass), lambda i: (0, 0)),
            pl.BlockSpec((2, nclass), lambda i: (0, 0)),
            pl.BlockSpec((1, nclass), lambda i: (0, 0)),
        ],
        out_specs=pl.BlockSpec((tm_b, nclass), lambda i: (i, 0)),
        out_shape=jax.ShapeDtypeStruct((n, nclass), jnp.float32),
        compiler_params=pltpu.CompilerParams(
            dimension_semantics=("arbitrary",),
        ),
    )(q, s2q, sc, b2r)


# speedup vs baseline: 1.0094x; 1.0079x over previous
import functools

import jax
import jax.numpy as jnp
from jax.experimental import pallas as pl
from jax.experimental.pallas import tpu as pltpu


def _pass_a_body(x_ref, adj_ref, W1_ref, b1_ref, W2_ref, q_ref, s2q_ref,
                 sc_ref, s1_ref, h_ref, *, tm, nt):
    i = pl.program_id(0)

    @pl.when(i == 0)
    def _():
        s1_ref[...] = jnp.dot(x_ref[...], W1_ref[...],
                              preferred_element_type=jnp.float32)

    a = adj_ref[...]
    acc = jnp.dot(a, s1_ref[...], preferred_element_type=jnp.float32)
    h_ref[pl.ds(i * tm, tm), :] = jnp.maximum(acc + b1_ref[...], 0.0)
    q_ref[...] = a.astype(jnp.float8_e4m3fn)

    @pl.when(i == nt - 1)
    def _():
        s2 = jnp.dot(h_ref[...], W2_ref[...],
                     preferred_element_type=jnp.float32)
        scale = jnp.maximum(jnp.max(jnp.abs(s2), axis=0, keepdims=True),
                            1e-30) * (1.0 / 240.0)
        s2q_ref[...] = (s2 / scale).astype(jnp.float8_e4m3fn)
        sc_ref[...] = scale


def _pass_b_body(q_ref, s2q_ref, sc_ref, b2_ref, out_ref):
    acc = jnp.dot(q_ref[...], s2q_ref[...],
                  preferred_element_type=jnp.float32)
    o = acc * sc_ref[...] + b2_ref[...]
    m = jnp.max(o, axis=1, keepdims=True)
    lse = jnp.log(jnp.sum(jnp.exp(o - m), axis=1, keepdims=True)) + m
    out_ref[...] = o - lse


def kernel(x, adj, W1, b1, W2, b2):
    n, nfeat = x.shape
    nhid = W1.shape[1]
    nclass = W2.shape[1]
    tm_a = 400
    nt_a = n // tm_a
    tm_b = 1000
    nt_b = n // tm_b

    b1r = b1.reshape(1, nhid)
    b2r = b2.reshape(1, nclass)

    q, s2q, sc = pl.pallas_call(
        functools.partial(_pass_a_body, tm=tm_a, nt=nt_a),
        grid=(nt_a,),
        in_specs=[
            pl.BlockSpec((n, nfeat), lambda i: (0, 0)),
            pl.BlockSpec((tm_a, n), lambda i: (i, 0)),
            pl.BlockSpec((nfeat, nhid), lambda i: (0, 0)),
            pl.BlockSpec((1, nhid), lambda i: (0, 0)),
            pl.BlockSpec((nhid, nclass), lambda i: (0, 0)),
        ],
        out_specs=[
            pl.BlockSpec((tm_a, n), lambda i: (i, 0)),
            pl.BlockSpec((n, nclass), lambda i: (0, 0)),
            pl.BlockSpec((1, nclass), lambda i: (0, 0)),
        ],
        out_shape=[
            jax.ShapeDtypeStruct((n, n), jnp.float8_e4m3fn),
            jax.ShapeDtypeStruct((n, nclass), jnp.float8_e4m3fn),
            jax.ShapeDtypeStruct((1, nclass), jnp.float32),
        ],
        scratch_shapes=[
            pltpu.VMEM((n, nhid), jnp.float32),
            pltpu.VMEM((n, nhid), jnp.float32),
        ],
        compiler_params=pltpu.CompilerParams(
            dimension_semantics=("arbitrary",),
            vmem_limit_bytes=63 * 1024 * 1024,
        ),
    )(x, adj, W1, b1r, W2)

    return pl.pallas_call(
        _pass_b_body,
        grid=(nt_b,),
        in_specs=[
            pl.BlockSpec((tm_b, n), lambda i: (i, 0)),
            pl.BlockSpec((n, nclass), lambda i: (0, 0)),
            pl.BlockSpec((1, nclass), lambda i: (0, 0)),
            pl.BlockSpec((1, nclass), lambda i: (0, 0)),
        ],
        out_specs=pl.BlockSpec((tm_b, nclass), lambda i: (i, 0)),
        out_shape=jax.ShapeDtypeStruct((n, nclass), jnp.float32),
        compiler_params=pltpu.CompilerParams(
            dimension_semantics=("arbitrary",),
        ),
    )(q, s2q, sc, b2r)
